# baseline (speedup 1.0000x reference)
import functools

import jax
import jax.numpy as jnp
from jax import lax
from jax.experimental import pallas as pl
from jax.experimental.pallas import tpu as pltpu
from jax.experimental.pallas import tpu_sc as plsc


def _prep_body(cb_ref, woutw_ref, cbna_ref, qt_ref, *, n_codebooks):
    cb = cb_ref[...]
    cbssq = jnp.sum(cb * cb, axis=2, keepdims=True)
    cbn = cb / jnp.maximum(jnp.sqrt(cbssq), 1e-12)
    chalf = jnp.sum(cbn * cbn, axis=2, keepdims=True) * 0.5
    cbna_ref[...] = jnp.concatenate([cbn, -chalf], axis=2)

    dn1 = (((1,), (1,)), ((), ()))
    for i in range(n_codebooks):
        q = jax.lax.dot_general(cb_ref[i], woutw_ref[i], dn1)
        qt_ref[i] = jnp.concatenate([q, jnp.zeros_like(q)], axis=1)


def _select_idx(res, winw_ref, winb_ref, cbna_ref, idx_ref, cb_size, tblk):
    dn = (((1,), (0,)), ((), ()))
    enc = jax.lax.dot_general(winw_ref[...], res, dn) + winb_ref[...]
    ssq = jnp.sum(enc * enc, axis=0, keepdims=True)
    encn = enc / jnp.maximum(jnp.sqrt(ssq), 1e-12)
    ones = jnp.ones((1, tblk), jnp.float32)
    encn_aug = jnp.concatenate([encn, ones], axis=0)
    s = jax.lax.dot_general(cbna_ref[...], encn_aug, dn)
    m = jnp.max(s, axis=0, keepdims=True)
    iota = jax.lax.broadcasted_iota(jnp.int32, (cb_size, tblk), 0)
    cand = jnp.where(s == m, iota, cb_size)
    idx_ref[0, 0] = jnp.min(cand, axis=0, keepdims=True)


def _stage0_body(mean_ref, scale_ref, noise_ref, winw_ref, winb_ref,
                 cbna_ref, res_ref, idx_ref, *, cb_size, tblk):
    stdev = jax.nn.softplus(scale_ref[0]) + 1e-4
    z = noise_ref[0] * stdev + mean_ref[0]
    res_ref[0] = z
    _select_idx(z, winw_ref, winb_ref, cbna_ref, idx_ref, cb_size, tblk)


def _mid_body(resp_ref, go_ref, woutbp_ref, winw_ref, winb_ref,
              cbna_ref, res_ref, idx_ref, *, cb_size, tblk):
    d = resp_ref.shape[1]
    got = jnp.transpose(go_ref[0, 0, :, 0:d], (1, 0))
    res = resp_ref[0] - (got + woutbp_ref[...])
    res_ref[0] = res
    _select_idx(res, winw_ref, winb_ref, cbna_ref, idx_ref, cb_size, tblk)


def _final_body(res0_ref, resp_ref, go_ref, woutbp_ref, out_ref):
    d = resp_ref.shape[1]
    got = jnp.transpose(go_ref[0, 0, :, 0:d], (1, 0))
    res_last = resp_ref[0] - (got + woutbp_ref[...])
    out_ref[0] = res0_ref[0] - res_last


def _make_sc_gather(nt, d, n_rows):
    info = plsc.get_sparse_core_info()
    nc, ns = info.num_cores, info.num_subcores
    nw = nc * ns
    per_w = nt // nw
    chunk = min(128, per_w)
    mesh = plsc.VectorSubcoreMesh(core_axis_name="c", subcore_axis_name="s")

    @functools.partial(
        pl.kernel, mesh=mesh,
        out_type=jax.ShapeDtypeStruct((nt, d), jnp.float32),
        scratch_types=[
            pltpu.VMEM((chunk,), jnp.int32),
            pltpu.VMEM((chunk, d), jnp.float32),
            pltpu.SemaphoreType.DMA,
        ],
    )
    def sc_gather(table_hbm, idx_hbm, out_hbm, idx_v, rows_v, sem):
        wid = lax.axis_index("s") * nc + lax.axis_index("c")
        base = wid * per_w
        for c in range(per_w // chunk):
            off = base + c * chunk
            pltpu.sync_copy(idx_hbm.at[pl.ds(off, chunk)], idx_v)
            pltpu.async_copy(table_hbm.at[idx_v], rows_v, sem).wait()
            pltpu.sync_copy(rows_v, out_hbm.at[pl.ds(off, chunk)])

    return sc_gather


def kernel(x, noise, in_proj_w, in_proj_b, out_proj_w, out_proj_b, codebooks):
    bsz, c2, t = x.shape
    d = c2 // 2
    ncb, cb_size, cd = codebooks.shape
    tblk = 512 if t % 512 == 0 else t
    nblk = t // tblk
    nt = bsz * t
    grid = (bsz, nblk)

    in_b = in_proj_b.reshape(ncb, cd, 1)
    out_b = out_proj_b.reshape(ncb, d, 1)

    cbna, qt = pl.pallas_call(
        functools.partial(_prep_body, n_codebooks=ncb),
        out_shape=[
            jax.ShapeDtypeStruct((ncb, cb_size, cd + 1), jnp.float32),
            jax.ShapeDtypeStruct((ncb, cb_size, 2 * d), jnp.float32),
        ],
    )(codebooks, out_proj_w)

    tok_spec = pl.BlockSpec((1, d, tblk), lambda b, tt: (b, 0, tt))
    idx_spec = pl.BlockSpec((1, 1, 1, tblk), lambda b, tt: (b, tt, 0, 0))
    go_spec = pl.BlockSpec((1, 1, tblk, 2 * d), lambda b, tt: (b, tt, 0, 0))
    w_spec = lambda shape: pl.BlockSpec(shape, lambda b, tt: (0,) * len(shape))
    res_shape = jax.ShapeDtypeStruct((bsz, d, t), jnp.float32)
    idx_shape = jax.ShapeDtypeStruct((bsz, nblk, 1, tblk), jnp.int32)

    sc_gather = _make_sc_gather(nt, 2 * d, cb_size)

    res0, idx = pl.pallas_call(
        functools.partial(_stage0_body, cb_size=cb_size, tblk=tblk),
        grid=grid,
        in_specs=[
            tok_spec,
            pl.BlockSpec((1, d, tblk), lambda b, tt: (b, 1, tt)),
            tok_spec,
            w_spec((cd, d)), w_spec((cd, 1)), w_spec((cb_size, cd + 1)),
        ],
        out_specs=[tok_spec, idx_spec],
        out_shape=[res_shape, idx_shape],
    )(x, x, noise, in_proj_w[0], in_b[0], cbna[0])

    res = res0
    for i in range(1, ncb + 1):
        go = sc_gather(qt[i - 1], idx.reshape(nt))
        go4 = go.reshape(bsz, nblk, tblk, 2 * d)
        if i < ncb:
            res, idx = pl.pallas_call(
                functools.partial(_mid_body, cb_size=cb_size, tblk=tblk),
                grid=grid,
                in_specs=[
                    tok_spec, go_spec, w_spec((d, 1)),
                    w_spec((cd, d)), w_spec((cd, 1)),
                    w_spec((cb_size, cd + 1)),
                ],
                out_specs=[tok_spec, idx_spec],
                out_shape=[res_shape, idx_shape],
            )(res, go4, out_b[i - 1], in_proj_w[i], in_b[i], cbna[i])
        else:
            return pl.pallas_call(
                _final_body,
                grid=grid,
                in_specs=[tok_spec, tok_spec, go_spec, w_spec((d, 1))],
                out_specs=tok_spec,
                out_shape=res_shape,
            )(res0, res, go4, out_b[i - 1])


# speedup vs baseline: 1.8672x; 1.8672x over previous
import functools

import jax
import jax.numpy as jnp
from jax.experimental import pallas as pl


def _prep_body(cb_ref, cbt_ref, woutw_ref, cbna_ref, q_ref, *, n_codebooks):
    cb = cb_ref[...]
    cbssq = jnp.sum(cb * cb, axis=2, keepdims=True)
    cbn = cb / jnp.maximum(jnp.sqrt(cbssq), 1e-12)
    chalf = jnp.sum(cbn * cbn, axis=2, keepdims=True) * 0.5
    cbna_ref[...] = jnp.concatenate([cbn, -chalf], axis=2)

    dn = (((1,), (0,)), ((), ()))
    for i in range(n_codebooks):
        q = jax.lax.dot_general(woutw_ref[i], cbt_ref[i], dn)
        hi = q.astype(jnp.bfloat16)
        r1 = q - hi.astype(jnp.float32)
        mid = r1.astype(jnp.bfloat16)
        lo = (r1 - mid.astype(jnp.float32)).astype(jnp.bfloat16)
        q_ref[i] = jnp.concatenate([hi, mid, lo], axis=0)


def _body(mean_ref, scale_ref, noise_ref, winw_ref, winb_ref, woutb_ref,
          cbna_ref, q_ref, out_ref, *, n_codebooks, tblk, d, nchain):
    mean = mean_ref[0]
    scale = scale_ref[0]
    noise = noise_ref[0]

    stdev = jax.nn.softplus(scale) + 1e-4
    z = noise * stdev + mean

    cw = tblk // nchain
    ones = jnp.ones((1, cw), jnp.float32)
    dn = (((1,), (0,)), ((), ()))
    f32 = jnp.float32

    res = [z[:, h * cw:(h + 1) * cw] for h in range(nchain)]
    acc = [jnp.zeros((d, cw), f32) for _ in range(nchain)]

    for i in range(n_codebooks):
        for h in range(nchain):
            enc = jax.lax.dot_general(winw_ref[i], res[h], dn) + winb_ref[i]
            ssq = jnp.sum(enc * enc, axis=0, keepdims=True)
            encn = enc / jnp.maximum(jnp.sqrt(ssq), 1e-12)
            encn_aug = jnp.concatenate([encn, ones], axis=0)

            s = jax.lax.dot_general(cbna_ref[i], encn_aug, dn)
            m = jnp.max(s, axis=0, keepdims=True)
            oh = (s == m).astype(jnp.bfloat16)

            g = jax.lax.dot_general(q_ref[i], oh, dn,
                                    preferred_element_type=f32)
            out = ((g[0:d] + g[d:2 * d]) + g[2 * d:3 * d]) + woutb_ref[i]
            acc[h] = acc[h] + out
            res[h] = res[h] - out

    out_ref[0] = jnp.concatenate(acc, axis=1)


def kernel(x, noise, in_proj_w, in_proj_b, out_proj_w, out_proj_b, codebooks):
    bsz, c2, t = x.shape
    d = c2 // 2
    ncb, cb_size, cd = codebooks.shape
    tblk = 2048 if t % 2048 == 0 else t
    nchain = 4 if tblk % 2048 == 0 else 1
    grid = (bsz, t // tblk)

    in_b = in_proj_b.reshape(ncb, cd, 1)
    out_b = out_proj_b.reshape(ncb, d, 1)
    cbt = jnp.transpose(codebooks, (0, 2, 1))

    cbna, q = pl.pallas_call(
        functools.partial(_prep_body, n_codebooks=ncb),
        out_shape=[
            jax.ShapeDtypeStruct((ncb, cb_size, cd + 1), jnp.float32),
            jax.ShapeDtypeStruct((ncb, 3 * d, cb_size), jnp.bfloat16),
        ],
    )(codebooks, cbt, out_proj_w)

    body = functools.partial(_body, n_codebooks=ncb, tblk=tblk, d=d,
                             nchain=nchain)

    tok_spec = pl.BlockSpec((1, d, tblk), lambda b, tt: (b, 0, tt))
    full = lambda shape: pl.BlockSpec(shape, lambda b, tt: (0,) * len(shape))

    return pl.pallas_call(
        body,
        grid=grid,
        in_specs=[
            tok_spec,
            pl.BlockSpec((1, d, tblk), lambda b, tt: (b, 1, tt)),
            tok_spec,
            full((ncb, cd, d)),
            full((ncb, cd, 1)),
            full((ncb, d, 1)),
            full((ncb, cb_size, cd + 1)),
            full((ncb, 3 * d, cb_size)),
        ],
        out_specs=tok_spec,
        out_shape=jax.ShapeDtypeStruct((bsz, d, t), jnp.float32),
    )(x, x, noise, in_proj_w, in_b, out_b, cbna, q)
